# Initial kernel scaffold; baseline (speedup 1.0000x reference)
import jax
import jax.numpy as jnp
from jax.experimental import pallas as pl


def kernel(seq_encoding, W_query, b_query, W_left, b_left, W_right, b_right, W_pos, b_pos, ln_q_g, ln_q_b, ln_p_g, ln_p_b, W_struc, b_struc, ln_out_g, ln_out_b):
    raise NotImplementedError("write your pallas kernel here")


# speedup vs baseline: 9.8158x; 9.8158x over previous
import functools

import jax
import jax.numpy as jnp
from jax.experimental import pallas as pl

N_ALPHA = 23
N_EMB_SEQ = 256
N_EMB_PAIR = 128
MAX_GAP = 32
N_INDEX = 2 * MAX_GAP + 1
APAD = 32
PPAD = 72
TI = 64


def _prep_body(idx_ref, wq_ref, wl_ref, wr_ref, wp_ref, cvec_ref, bq_ref,
               bl_ref, br_ref, seq_ref, xl_ref, xr_ref, e_ref, *, BL, L):
    idx = idx_ref[...]
    cols = jax.lax.broadcasted_iota(jnp.int32, (BL, APAD), 1)
    oh = (idx == cols).astype(jnp.float32)
    seq = jax.lax.dot(oh, wq_ref[...])
    seq_ref[...] = seq + bq_ref[...]
    xl_ref[...] = jax.lax.dot(oh, wl_ref[...]) + bl_ref[...]
    xr_ref[...] = jax.lax.dot(oh, wr_ref[...]) + br_ref[...]
    k = jax.lax.broadcasted_iota(jnp.int32, (BL, PPAD), 0)
    m = jnp.clip(k - (L - 1), -MAX_GAP, MAX_GAP) + MAX_GAP
    pcols = jax.lax.broadcasted_iota(jnp.int32, (BL, PPAD), 1)
    ohp = (m == pcols).astype(jnp.float32)
    e_ref[...] = jax.lax.dot(ohp, wp_ref[...]) + cvec_ref[...]


def _pair_body(xl_ref, xr_ref, e_ref, g_ref, b_ref, out_ref, *, L):
    i0 = pl.program_id(1) * TI
    xr = xr_ref[0]
    g = g_ref[...]
    bb = b_ref[...]

    def row(ti, carry):
        xli = xl_ref[0, ti, :]
        start = (L - 1) - (i0 + ti)
        esl = e_ref[pl.ds(start, L), :]
        v = xli[None, :] + xr + esl
        m = jnp.mean(v, axis=-1, keepdims=True)
        var = jnp.mean((v - m) * (v - m), axis=-1, keepdims=True)
        out_ref[0, ti] = (v - m) * jax.lax.rsqrt(var + 1e-5) * g + bb
        return carry

    jax.lax.fori_loop(0, TI, row, 0, unroll=False)


def kernel(seq_encoding, W_query, b_query, W_left, b_left, W_right, b_right,
           W_pos, b_pos, ln_q_g, ln_q_b, ln_p_g, ln_p_b, W_struc, b_struc,
           ln_out_g, ln_out_b):
    B, L = seq_encoding.shape
    BL = B * L
    f32 = jnp.float32

    wq = jnp.zeros((APAD, N_EMB_SEQ), f32).at[:N_ALPHA].set(W_query)
    wl = jnp.zeros((APAD, N_EMB_PAIR), f32).at[:N_ALPHA].set(W_left)
    wr = jnp.zeros((APAD, N_EMB_PAIR), f32).at[:N_ALPHA].set(W_right)
    wp = jnp.zeros((PPAD, N_EMB_PAIR), f32).at[:N_INDEX].set(W_pos)
    cvec = (b_pos + ln_p_b + W_struc[0] + b_struc).reshape(1, N_EMB_PAIR)
    bq = (b_query + ln_q_b).reshape(1, N_EMB_SEQ)
    blv = b_left.reshape(1, N_EMB_PAIR)
    brv = b_right.reshape(1, N_EMB_PAIR)
    idx = seq_encoding.astype(jnp.int32).reshape(BL, 1)

    seq_flat, xl, xr, etab = pl.pallas_call(
        functools.partial(_prep_body, BL=BL, L=L),
        out_shape=(
            jax.ShapeDtypeStruct((BL, N_EMB_SEQ), f32),
            jax.ShapeDtypeStruct((BL, N_EMB_PAIR), f32),
            jax.ShapeDtypeStruct((BL, N_EMB_PAIR), f32),
            jax.ShapeDtypeStruct((BL, N_EMB_PAIR), f32),
        ),
    )(idx, wq, wl, wr, wp, cvec, bq, blv, brv)

    xl3 = xl.reshape(B, L, N_EMB_PAIR)
    xr3 = xr.reshape(B, L, N_EMB_PAIR)
    g2 = ln_out_g.reshape(1, N_EMB_PAIR)
    b2 = ln_out_b.reshape(1, N_EMB_PAIR)

    pair = pl.pallas_call(
        functools.partial(_pair_body, L=L),
        grid=(B, L // TI),
        in_specs=[
            pl.BlockSpec((1, TI, N_EMB_PAIR), lambda b, i: (b, i, 0)),
            pl.BlockSpec((1, L, N_EMB_PAIR), lambda b, i: (b, 0, 0)),
            pl.BlockSpec((BL, N_EMB_PAIR), lambda b, i: (0, 0)),
            pl.BlockSpec((1, N_EMB_PAIR), lambda b, i: (0, 0)),
            pl.BlockSpec((1, N_EMB_PAIR), lambda b, i: (0, 0)),
        ],
        out_specs=pl.BlockSpec((1, TI, L, N_EMB_PAIR),
                               lambda b, i: (b, i, 0, 0)),
        out_shape=jax.ShapeDtypeStruct((B, L, L, N_EMB_PAIR), f32),
    )(xl3, xr3, etab, g2, b2)

    seq_out = seq_flat.reshape(B, 1, L, N_EMB_SEQ)
    return (seq_out, pair)


# speedup vs baseline: 14.4489x; 1.4720x over previous
import functools

import jax
import jax.numpy as jnp
from jax.experimental import pallas as pl

N_ALPHA = 23
N_EMB_SEQ = 256
N_EMB_PAIR = 128
MAX_GAP = 32
N_INDEX = 2 * MAX_GAP + 1
APAD = 32
PPAD = 72
TI = 64


def _prep_body(idx_ref, wq_ref, wl_ref, wr_ref, wp_ref, cvec_ref, bq_ref,
               bl_ref, br_ref, seq_ref, xl_ref, xr_ref, e_ref, *, BL, L):
    idx = idx_ref[...]
    cols = jax.lax.broadcasted_iota(jnp.int32, (BL, APAD), 1)
    oh = (idx == cols).astype(jnp.float32)
    seq = jax.lax.dot(oh, wq_ref[...])
    seq_ref[...] = seq + bq_ref[...]
    xl_ref[...] = jax.lax.dot(oh, wl_ref[...]) + bl_ref[...]
    xr_ref[...] = jax.lax.dot(oh, wr_ref[...]) + br_ref[...]
    k = jax.lax.broadcasted_iota(jnp.int32, (BL, PPAD), 0)
    m = jnp.clip(k - (L - 1), -MAX_GAP, MAX_GAP) + MAX_GAP
    pcols = jax.lax.broadcasted_iota(jnp.int32, (BL, PPAD), 1)
    ohp = (m == pcols).astype(jnp.float32)
    e_ref[...] = jax.lax.dot(ohp, wp_ref[...]) + cvec_ref[...]


def _pair_body(xl_ref, xr_ref, e_ref, g_ref, b_ref, out_ref, *, L):
    i0 = pl.program_id(1) * TI
    xr = xr_ref[0]
    g = g_ref[...]
    bb = b_ref[...]

    def row(ti, carry):
        xli = xl_ref[0, ti, :]
        start = (L - 1) - (i0 + ti)
        esl = e_ref[pl.ds(start, L), :]
        v = xli[None, :] + xr + esl
        m = jnp.mean(v, axis=-1, keepdims=True)
        var = jnp.mean((v - m) * (v - m), axis=-1, keepdims=True)
        out_ref[0, ti] = (v - m) * jax.lax.rsqrt(var + 1e-5) * g + bb
        return carry

    jax.lax.fori_loop(0, TI, row, 0, unroll=8)


def kernel(seq_encoding, W_query, b_query, W_left, b_left, W_right, b_right,
           W_pos, b_pos, ln_q_g, ln_q_b, ln_p_g, ln_p_b, W_struc, b_struc,
           ln_out_g, ln_out_b):
    B, L = seq_encoding.shape
    BL = B * L
    f32 = jnp.float32

    wq = jnp.zeros((APAD, N_EMB_SEQ), f32).at[:N_ALPHA].set(W_query)
    wl = jnp.zeros((APAD, N_EMB_PAIR), f32).at[:N_ALPHA].set(W_left)
    wr = jnp.zeros((APAD, N_EMB_PAIR), f32).at[:N_ALPHA].set(W_right)
    wp = jnp.zeros((PPAD, N_EMB_PAIR), f32).at[:N_INDEX].set(W_pos)
    cvec = (b_pos + ln_p_b + W_struc[0] + b_struc).reshape(1, N_EMB_PAIR)
    bq = (b_query + ln_q_b).reshape(1, N_EMB_SEQ)
    blv = b_left.reshape(1, N_EMB_PAIR)
    brv = b_right.reshape(1, N_EMB_PAIR)
    idx = seq_encoding.astype(jnp.int32).reshape(BL, 1)

    seq_flat, xl, xr, etab = pl.pallas_call(
        functools.partial(_prep_body, BL=BL, L=L),
        out_shape=(
            jax.ShapeDtypeStruct((BL, N_EMB_SEQ), f32),
            jax.ShapeDtypeStruct((BL, N_EMB_PAIR), f32),
            jax.ShapeDtypeStruct((BL, N_EMB_PAIR), f32),
            jax.ShapeDtypeStruct((BL, N_EMB_PAIR), f32),
        ),
    )(idx, wq, wl, wr, wp, cvec, bq, blv, brv)

    xl3 = xl.reshape(B, L, N_EMB_PAIR)
    xr3 = xr.reshape(B, L, N_EMB_PAIR)
    g2 = ln_out_g.reshape(1, N_EMB_PAIR)
    b2 = ln_out_b.reshape(1, N_EMB_PAIR)

    pair = pl.pallas_call(
        functools.partial(_pair_body, L=L),
        grid=(B, L // TI),
        in_specs=[
            pl.BlockSpec((1, TI, N_EMB_PAIR), lambda b, i: (b, i, 0)),
            pl.BlockSpec((1, L, N_EMB_PAIR), lambda b, i: (b, 0, 0)),
            pl.BlockSpec((BL, N_EMB_PAIR), lambda b, i: (0, 0)),
            pl.BlockSpec((1, N_EMB_PAIR), lambda b, i: (0, 0)),
            pl.BlockSpec((1, N_EMB_PAIR), lambda b, i: (0, 0)),
        ],
        out_specs=pl.BlockSpec((1, TI, L, N_EMB_PAIR),
                               lambda b, i: (b, i, 0, 0)),
        out_shape=jax.ShapeDtypeStruct((B, L, L, N_EMB_PAIR), f32),
    )(xl3, xr3, etab, g2, b2)

    seq_out = seq_flat.reshape(B, 1, L, N_EMB_SEQ)
    return (seq_out, pair)
